# Initial kernel scaffold; baseline (speedup 1.0000x reference)
import jax
import jax.numpy as jnp
from jax.experimental import pallas as pl


def kernel(x, edge_index, W1, b1, W2, b2):
    raise NotImplementedError("write your pallas kernel here")


# speedup vs baseline: 6.3002x; 6.3002x over previous
import functools

import jax
import jax.numpy as jnp
from jax import lax
from jax.experimental import pallas as pl
from jax.experimental.pallas import tpu as pltpu
from jax.experimental.pallas import tpu_sc as plsc

N = 10000
E = 160000
NC = 2
NS = 16
L = 16
RS = 640
B = 80
EPS = E // NS
NB = EPS // B
BD = 40
EPW = E // (NC * NS)
NBD = EPW // BD
MB = 1000
NBN = N // MB

_f32 = jnp.float32
_mesh = plsc.VectorSubcoreMesh(core_axis_name="c", subcore_axis_name="s")


@functools.partial(
    pl.kernel,
    out_type=jax.ShapeDtypeStruct((NC * N, 16), _f32),
    mesh=_mesh,
    scratch_types=[
        pltpu.VMEM_SHARED((N, 16), _f32),
        pltpu.VMEM((BD,), jnp.int32),
        pltpu.VMEM((RS, 16), _f32),
    ],
)
def _deg_kernel(dst_hbm, out_hbm, acc, dstv, ones_v):
    c = lax.axis_index("c")
    s = lax.axis_index("s")

    @pl.loop(0, RS)
    def _(i):
        ones_v[i, :] = jnp.ones((L,), _f32)

    @pl.when(s < NS - 1)
    def _():
        pltpu.sync_copy(ones_v, acc.at[pl.ds(s * RS, RS)])

    @pl.when(s == NS - 1)
    def _():
        pltpu.sync_copy(ones_v.at[pl.ds(0, N - (NS - 1) * RS)],
                        acc.at[pl.ds((NS - 1) * RS, N - (NS - 1) * RS)])

    plsc.subcore_barrier()

    base = (s * NC + c) * EPW

    @pl.loop(0, NBD)
    def _(bk):
        pltpu.sync_copy(dst_hbm.at[pl.ds(base + bk * BD, BD)], dstv)
        pltpu.sync_copy(ones_v.at[pl.ds(0, BD)], acc.at[dstv], add=True)

    plsc.subcore_barrier()

    @pl.when(s < NS - 1)
    def _():
        pltpu.sync_copy(acc.at[pl.ds(s * RS, RS)],
                        out_hbm.at[pl.ds(c * N + s * RS, RS)])

    @pl.when(s == NS - 1)
    def _():
        pltpu.sync_copy(acc.at[pl.ds((NS - 1) * RS, N - (NS - 1) * RS)],
                        out_hbm.at[pl.ds(c * N + (NS - 1) * RS,
                                         N - (NS - 1) * RS)])


def _make_agg_kernel(C):
    @functools.partial(
        pl.kernel,
        out_type=jax.ShapeDtypeStruct((C * N, 128), _f32),
        mesh=_mesh,
        scratch_types=[
            pltpu.VMEM_SHARED((N, 128), _f32),
            pltpu.VMEM((B,), jnp.int32),
            pltpu.VMEM((B,), jnp.int32),
            pltpu.VMEM((B, 128), _f32),
        ],
    )
    def _agg(hs_hbm, src_hbm, dst_hbm, out_hbm, acc, srcv, dstv, rows):
        c = lax.axis_index("c")
        s = lax.axis_index("s")
        for p in range(C // NC):
            chunk = c + NC * p
            row0 = chunk * N
            @pl.when(s < NS - 1)
            def _():
                pltpu.sync_copy(hs_hbm.at[pl.ds(row0 + s * RS, RS)],
                                acc.at[pl.ds(s * RS, RS)])

            @pl.when(s == NS - 1)
            def _():
                tail = N - (NS - 1) * RS
                pltpu.sync_copy(hs_hbm.at[pl.ds(row0 + (NS - 1) * RS, tail)],
                                acc.at[pl.ds((NS - 1) * RS, tail)])

            plsc.subcore_barrier()

            base = s * EPS

            @pl.loop(0, NB)
            def _(bk):
                off = base + bk * B
                pltpu.sync_copy(src_hbm.at[pl.ds(off, B)], srcv)
                pltpu.sync_copy(dst_hbm.at[pl.ds(off, B)], dstv)
                for j in range(B // L):
                    sl = pl.ds(j * L, L)
                    srcv[sl] = srcv[sl] + row0
                pltpu.sync_copy(hs_hbm.at[srcv], rows)
                pltpu.sync_copy(rows, acc.at[dstv], add=True)

            plsc.subcore_barrier()

            @pl.when(s < NS - 1)
            def _():
                pltpu.sync_copy(acc.at[pl.ds(s * RS, RS)],
                                out_hbm.at[pl.ds(row0 + s * RS, RS)])

            @pl.when(s == NS - 1)
            def _():
                tail = N - (NS - 1) * RS
                pltpu.sync_copy(acc.at[pl.ds((NS - 1) * RS, tail)],
                                out_hbm.at[pl.ds(row0 + (NS - 1) * RS, tail)])

            if p + 1 < C // NC:
                plsc.subcore_barrier()

    return _agg


_agg2 = _make_agg_kernel(2)
_agg4 = _make_agg_kernel(4)


def _dinv_call(degp):
    def body(p_ref, o_ref):
        deg = p_ref[0:N, :] + p_ref[N:2 * N, :] - 1.0
        o_ref[...] = jax.lax.rsqrt(jnp.concatenate([deg] * 8, axis=1))

    return pl.pallas_call(
        body, out_shape=jax.ShapeDtypeStruct((N, 128), _f32))(degp)


def _mm_chunked(xin, W, dinv, C):
    K = xin.shape[1]

    def body(x_ref, w_ref, d_ref, o_ref):
        o_ref[...] = jnp.dot(x_ref[...], w_ref[...],
                             preferred_element_type=_f32) * d_ref[...]

    return pl.pallas_call(
        body,
        grid=(NBN, C),
        in_specs=[
            pl.BlockSpec((MB, K), lambda i, c: (i, 0)),
            pl.BlockSpec((K, 128), lambda i, c: (0, c)),
            pl.BlockSpec((MB, 128), lambda i, c: (i, 0)),
        ],
        out_specs=pl.BlockSpec((MB, 128), lambda i, c: (c * NBN + i, 0)),
        out_shape=jax.ShapeDtypeStruct((C * N, 128), _f32),
    )(xin, W, dinv)


def _epi_call(agg, dinv, b, C, relu):
    def body(a_ref, d_ref, b_ref, o_ref):
        r = a_ref[...] * d_ref[...] + b_ref[0]
        o_ref[...] = jnp.maximum(r, 0.0) if relu else r

    return pl.pallas_call(
        body,
        grid=(NBN, C),
        in_specs=[
            pl.BlockSpec((MB, 128), lambda i, c: (c * NBN + i, 0)),
            pl.BlockSpec((MB, 128), lambda i, c: (i, 0)),
            pl.BlockSpec((1, 1, 128), lambda i, c: (c, 0, 0)),
        ],
        out_specs=pl.BlockSpec((MB, 128), lambda i, c: (i, c)),
        out_shape=jax.ShapeDtypeStruct((N, C * 128), _f32),
    )(agg, dinv, b.reshape(C, 1, 128))


def kernel(x, edge_index, W1, b1, W2, b2):
    ei = edge_index.astype(jnp.int32)
    src = ei[0]
    dst = ei[1]

    degp = _deg_kernel(dst)
    dinv = _dinv_call(degp)

    hs1 = _mm_chunked(x, W1, dinv, 4)
    agg1 = _agg4(hs1, src, dst)
    out1 = _epi_call(agg1, dinv, b1, 4, relu=True)

    hs2 = _mm_chunked(out1, W2, dinv, 2)
    agg2 = _agg2(hs2, src, dst)
    out2 = _epi_call(agg2, dinv, b2, 2, relu=False)

    return (out2, x, out1, out2)


# speedup vs baseline: 12.3601x; 1.9619x over previous
import functools

import jax
import jax.numpy as jnp
from jax import lax
from jax.experimental import pallas as pl
from jax.experimental.pallas import tpu as pltpu
from jax.experimental.pallas import tpu_sc as plsc

N = 10000
E = 160000
NC = 2
NS = 16
L = 16
RS = 640
B = 80
EPS = E // NS
NB = EPS // B
BD = 40
EPW = E // (NC * NS)
NBD = EPW // BD
MB = 1000
NBN = N // MB

_f32 = jnp.float32
_mesh = plsc.VectorSubcoreMesh(core_axis_name="c", subcore_axis_name="s")


@functools.partial(
    pl.kernel,
    out_type=jax.ShapeDtypeStruct((NC * N, 16), _f32),
    mesh=_mesh,
    scratch_types=[
        pltpu.VMEM_SHARED((N, 16), _f32),
        pltpu.VMEM((BD,), jnp.int32),
        pltpu.VMEM((RS, 16), _f32),
    ],
)
def _deg_kernel(dst_hbm, out_hbm, acc, dstv, ones_v):
    c = lax.axis_index("c")
    s = lax.axis_index("s")

    @pl.loop(0, RS)
    def _(i):
        ones_v[i, :] = jnp.ones((L,), _f32)

    @pl.when(s < NS - 1)
    def _():
        pltpu.sync_copy(ones_v, acc.at[pl.ds(s * RS, RS)])

    @pl.when(s == NS - 1)
    def _():
        pltpu.sync_copy(ones_v.at[pl.ds(0, N - (NS - 1) * RS)],
                        acc.at[pl.ds((NS - 1) * RS, N - (NS - 1) * RS)])

    plsc.subcore_barrier()

    base = (s * NC + c) * EPW

    @pl.loop(0, NBD)
    def _(bk):
        pltpu.sync_copy(dst_hbm.at[pl.ds(base + bk * BD, BD)], dstv)
        pltpu.sync_copy(ones_v.at[pl.ds(0, BD)], acc.at[dstv], add=True)

    plsc.subcore_barrier()

    @pl.when(s < NS - 1)
    def _():
        pltpu.sync_copy(acc.at[pl.ds(s * RS, RS)],
                        out_hbm.at[pl.ds(c * N + s * RS, RS)])

    @pl.when(s == NS - 1)
    def _():
        pltpu.sync_copy(acc.at[pl.ds((NS - 1) * RS, N - (NS - 1) * RS)],
                        out_hbm.at[pl.ds(c * N + (NS - 1) * RS,
                                         N - (NS - 1) * RS)])


def _make_agg_kernel(C):
    @functools.partial(
        pl.kernel,
        out_type=jax.ShapeDtypeStruct((C * N, 128), _f32),
        mesh=_mesh,
        scratch_types=[
            pltpu.VMEM_SHARED((N, 128), _f32),
            pltpu.VMEM((NB, B), jnp.int32),
            pltpu.VMEM((B,), jnp.int32),
            pltpu.VMEM((B,), jnp.int32),
            pltpu.VMEM((B, 128), _f32),
            pltpu.VMEM((B, 128), _f32),
            pltpu.SemaphoreType.DMA,
            pltpu.SemaphoreType.DMA,
        ],
    )
    def _agg(hs_hbm, src_hbm, dst_hbm, out_hbm, acc, srcv, dstb0, dstb1,
             rows0, rows1, sem0, sem1):
        c = lax.axis_index("c")
        s = lax.axis_index("s")

        pltpu.sync_copy(src_hbm.at[s], srcv)

        def _shift(delta):
            @pl.loop(0, NB)
            def _(i):
                for j in range(B // L):
                    sl = pl.ds(j * L, L)
                    srcv[i, sl] = srcv[i, sl] + delta

        _shift(c * N)

        def _gather(b, rbuf, dbuf, sem):
            pltpu.async_copy(hs_hbm.at[srcv.at[b]], rbuf, sem)
            pltpu.async_copy(dst_hbm.at[pl.ds(s * EPS + b * B, B)], dbuf, sem)

        def _drain(b, rbuf, dbuf, sem):
            pltpu.make_async_copy(hs_hbm.at[srcv.at[b]], rbuf, sem).wait()
            pltpu.make_async_copy(dst_hbm.at[pl.ds(s * EPS + b * B, B)],
                                  dbuf, sem).wait()
            pltpu.sync_copy(rbuf, acc.at[dbuf], add=True)

        for p in range(C // NC):
            chunk = c + NC * p
            row0 = chunk * N
            if p > 0:
                _shift(NC * N)
            @pl.when(s < NS - 1)
            def _():
                pltpu.sync_copy(hs_hbm.at[pl.ds(row0 + s * RS, RS)],
                                acc.at[pl.ds(s * RS, RS)])

            @pl.when(s == NS - 1)
            def _():
                tail = N - (NS - 1) * RS
                pltpu.sync_copy(hs_hbm.at[pl.ds(row0 + (NS - 1) * RS, tail)],
                                acc.at[pl.ds((NS - 1) * RS, tail)])

            plsc.subcore_barrier()

            _gather(0, rows0, dstb0, sem0)

            @pl.loop(0, (NB - 1) // 2)
            def _(k):
                b = 2 * k
                _gather(b + 1, rows1, dstb1, sem1)
                _drain(b, rows0, dstb0, sem0)
                _gather(b + 2, rows0, dstb0, sem0)
                _drain(b + 1, rows1, dstb1, sem1)

            _drain(NB - 1, rows0, dstb0, sem0)

            plsc.subcore_barrier()

            @pl.when(s < NS - 1)
            def _():
                pltpu.sync_copy(acc.at[pl.ds(s * RS, RS)],
                                out_hbm.at[pl.ds(row0 + s * RS, RS)])

            @pl.when(s == NS - 1)
            def _():
                tail = N - (NS - 1) * RS
                pltpu.sync_copy(acc.at[pl.ds((NS - 1) * RS, tail)],
                                out_hbm.at[pl.ds(row0 + (NS - 1) * RS, tail)])

            if p + 1 < C // NC:
                plsc.subcore_barrier()

    return _agg


_agg2 = _make_agg_kernel(2)
_agg4 = _make_agg_kernel(4)


def _dinv_call(degp):
    def body(p_ref, o_ref):
        deg = p_ref[0:N, :] + p_ref[N:2 * N, :] - 1.0
        o_ref[...] = jax.lax.rsqrt(jnp.concatenate([deg] * 8, axis=1))

    return pl.pallas_call(
        body, out_shape=jax.ShapeDtypeStruct((N, 128), _f32))(degp)


def _mm_chunked(xin, W, dinv, C):
    K = xin.shape[1]

    def body(x_ref, w_ref, d_ref, o_ref):
        o_ref[...] = jnp.dot(x_ref[...], w_ref[...],
                             preferred_element_type=_f32) * d_ref[...]

    return pl.pallas_call(
        body,
        grid=(NBN, C),
        in_specs=[
            pl.BlockSpec((MB, K), lambda i, c: (i, 0)),
            pl.BlockSpec((K, 128), lambda i, c: (0, c)),
            pl.BlockSpec((MB, 128), lambda i, c: (i, 0)),
        ],
        out_specs=pl.BlockSpec((MB, 128), lambda i, c: (c * NBN + i, 0)),
        out_shape=jax.ShapeDtypeStruct((C * N, 128), _f32),
    )(xin, W, dinv)


def _epi_call(agg, dinv, b, C, relu):
    def body(a_ref, d_ref, b_ref, o_ref):
        r = a_ref[...] * d_ref[...] + b_ref[0]
        o_ref[...] = jnp.maximum(r, 0.0) if relu else r

    return pl.pallas_call(
        body,
        grid=(NBN, C),
        in_specs=[
            pl.BlockSpec((MB, 128), lambda i, c: (c * NBN + i, 0)),
            pl.BlockSpec((MB, 128), lambda i, c: (i, 0)),
            pl.BlockSpec((1, 1, 128), lambda i, c: (c, 0, 0)),
        ],
        out_specs=pl.BlockSpec((MB, 128), lambda i, c: (i, c)),
        out_shape=jax.ShapeDtypeStruct((N, C * 128), _f32),
    )(agg, dinv, b.reshape(C, 1, 128))


def kernel(x, edge_index, W1, b1, W2, b2):
    ei = edge_index.astype(jnp.int32)
    src = ei[0]
    dst = ei[1]

    src3 = src.reshape(NS, NB, B)

    degp = _deg_kernel(dst)
    dinv = _dinv_call(degp)

    hs1 = _mm_chunked(x, W1, dinv, 4)
    agg1 = _agg4(hs1, src3, dst)
    out1 = _epi_call(agg1, dinv, b1, 4, relu=True)

    hs2 = _mm_chunked(out1, W2, dinv, 2)
    agg2 = _agg2(hs2, src3, dst)
    out2 = _epi_call(agg2, dinv, b2, 2, relu=False)

    return (out2, x, out1, out2)


# speedup vs baseline: 13.2825x; 1.0746x over previous
import functools

import jax
import jax.numpy as jnp
from jax import lax
from jax.experimental import pallas as pl
from jax.experimental.pallas import tpu as pltpu
from jax.experimental.pallas import tpu_sc as plsc

N = 10000
E = 160000
NC = 2
NS = 16
L = 16
RS = 640
B = 80
EPS = E // NS
NB = EPS // B
BD = 40
BD2 = 80
NBD2 = 62
EPW = E // (NC * NS)
MB = 1000
NBN = N // MB

_f32 = jnp.float32
_mesh = plsc.VectorSubcoreMesh(core_axis_name="c", subcore_axis_name="s")


@functools.partial(
    pl.kernel,
    out_type=jax.ShapeDtypeStruct((NC * N, 16), _f32),
    mesh=_mesh,
    scratch_types=[
        pltpu.VMEM_SHARED((N, 16), _f32),
        pltpu.VMEM((BD2,), jnp.int32),
        pltpu.VMEM((BD2,), jnp.int32),
        pltpu.VMEM((BD,), jnp.int32),
        pltpu.VMEM((BD2, 16), _f32),
        pltpu.SemaphoreType.DMA,
        pltpu.SemaphoreType.DMA,
    ],
)
def _deg_kernel(dst_hbm, zeros_hbm, out_hbm, acc, dstb0, dstb1, dstbt,
                ones_v, sem0, sem1):
    c = lax.axis_index("c")
    s = lax.axis_index("s")

    @pl.loop(0, BD2)
    def _(i):
        ones_v[i, :] = jnp.ones((L,), _f32)

    @pl.when(s < NS - 1)
    def _():
        pltpu.sync_copy(zeros_hbm.at[pl.ds(s * RS, RS)],
                        acc.at[pl.ds(s * RS, RS)])

    @pl.when(s == NS - 1)
    def _():
        tail = N - (NS - 1) * RS
        pltpu.sync_copy(zeros_hbm.at[pl.ds((NS - 1) * RS, tail)],
                        acc.at[pl.ds((NS - 1) * RS, tail)])

    plsc.subcore_barrier()

    base = (s * NC + c) * EPW

    def _load(bk, dbuf, sem):
        pltpu.async_copy(dst_hbm.at[pl.ds(base + bk * BD2, BD2)], dbuf, sem)

    def _drain(bk, dbuf, sem):
        pltpu.make_async_copy(dst_hbm.at[pl.ds(base + bk * BD2, BD2)],
                              dbuf, sem).wait()
        pltpu.sync_copy(ones_v, acc.at[dbuf], add=True)

    _load(0, dstb0, sem0)

    @pl.loop(0, NBD2 // 2 - 1)
    def _(k):
        bk = 2 * k
        _load(bk + 1, dstb1, sem1)
        _drain(bk, dstb0, sem0)
        _load(bk + 2, dstb0, sem0)
        _drain(bk + 1, dstb1, sem1)

    _load(NBD2 - 1, dstb1, sem1)
    _drain(NBD2 - 2, dstb0, sem0)
    _drain(NBD2 - 1, dstb1, sem1)

    pltpu.sync_copy(dst_hbm.at[pl.ds(base + NBD2 * BD2, BD)], dstbt)
    pltpu.sync_copy(ones_v.at[pl.ds(0, BD)], acc.at[dstbt], add=True)

    plsc.subcore_barrier()

    @pl.when(s < NS - 1)
    def _():
        pltpu.sync_copy(acc.at[pl.ds(s * RS, RS)],
                        out_hbm.at[pl.ds(c * N + s * RS, RS)])

    @pl.when(s == NS - 1)
    def _():
        pltpu.sync_copy(acc.at[pl.ds((NS - 1) * RS, N - (NS - 1) * RS)],
                        out_hbm.at[pl.ds(c * N + (NS - 1) * RS,
                                         N - (NS - 1) * RS)])


def _make_agg_kernel(C):
    @functools.partial(
        pl.kernel,
        out_type=jax.ShapeDtypeStruct((C * N, 128), _f32),
        mesh=_mesh,
        scratch_types=[
            pltpu.VMEM_SHARED((N, 128), _f32),
            pltpu.VMEM((NB, B), jnp.int32),
            pltpu.VMEM((B,), jnp.int32),
            pltpu.VMEM((B,), jnp.int32),
            pltpu.VMEM((B, 128), _f32),
            pltpu.VMEM((B, 128), _f32),
            pltpu.SemaphoreType.DMA,
            pltpu.SemaphoreType.DMA,
        ],
    )
    def _agg(hs_hbm, src_hbm, dst_hbm, out_hbm, acc, srcv, dstb0, dstb1,
             rows0, rows1, sem0, sem1):
        c = lax.axis_index("c")
        s = lax.axis_index("s")

        pltpu.sync_copy(src_hbm.at[s], srcv)

        def _shift(delta):
            @pl.loop(0, NB)
            def _(i):
                for j in range(B // L):
                    sl = pl.ds(j * L, L)
                    srcv[i, sl] = srcv[i, sl] + delta

        _shift(c * N)

        def _gather(b, rbuf, dbuf, sem):
            pltpu.async_copy(hs_hbm.at[srcv.at[b]], rbuf, sem)
            pltpu.async_copy(dst_hbm.at[pl.ds(s * EPS + b * B, B)], dbuf, sem)

        def _drain(b, rbuf, dbuf, sem):
            pltpu.make_async_copy(hs_hbm.at[srcv.at[b]], rbuf, sem).wait()
            pltpu.make_async_copy(dst_hbm.at[pl.ds(s * EPS + b * B, B)],
                                  dbuf, sem).wait()
            pltpu.sync_copy(rbuf, acc.at[dbuf], add=True)

        for p in range(C // NC):
            chunk = c + NC * p
            row0 = chunk * N
            if p > 0:
                _shift(NC * N)
            @pl.when(s < NS - 1)
            def _():
                pltpu.sync_copy(hs_hbm.at[pl.ds(row0 + s * RS, RS)],
                                acc.at[pl.ds(s * RS, RS)])

            @pl.when(s == NS - 1)
            def _():
                tail = N - (NS - 1) * RS
                pltpu.sync_copy(hs_hbm.at[pl.ds(row0 + (NS - 1) * RS, tail)],
                                acc.at[pl.ds((NS - 1) * RS, tail)])

            plsc.subcore_barrier()

            _gather(0, rows0, dstb0, sem0)

            @pl.loop(0, (NB - 1) // 2)
            def _(k):
                b = 2 * k
                _gather(b + 1, rows1, dstb1, sem1)
                _drain(b, rows0, dstb0, sem0)
                _gather(b + 2, rows0, dstb0, sem0)
                _drain(b + 1, rows1, dstb1, sem1)

            _drain(NB - 1, rows0, dstb0, sem0)

            plsc.subcore_barrier()

            @pl.when(s < NS - 1)
            def _():
                pltpu.sync_copy(acc.at[pl.ds(s * RS, RS)],
                                out_hbm.at[pl.ds(row0 + s * RS, RS)])

            @pl.when(s == NS - 1)
            def _():
                tail = N - (NS - 1) * RS
                pltpu.sync_copy(acc.at[pl.ds((NS - 1) * RS, tail)],
                                out_hbm.at[pl.ds(row0 + (NS - 1) * RS, tail)])

            if p + 1 < C // NC:
                plsc.subcore_barrier()

    return _agg


_agg2 = _make_agg_kernel(2)
_agg4 = _make_agg_kernel(4)


def _dinv_call(degp):
    def body(p_ref, o_ref):
        deg = p_ref[0:N, :] + p_ref[N:2 * N, :] + 1.0
        o_ref[...] = jax.lax.rsqrt(jnp.concatenate([deg] * 8, axis=1))

    return pl.pallas_call(
        body, out_shape=jax.ShapeDtypeStruct((N, 128), _f32))(degp)


def _mm_chunked(xin, W, dinv, C):
    K = xin.shape[1]

    def body(x_ref, w_ref, d_ref, o_ref):
        o_ref[...] = jnp.dot(x_ref[...], w_ref[...],
                             preferred_element_type=_f32) * d_ref[...]

    return pl.pallas_call(
        body,
        grid=(NBN, C),
        in_specs=[
            pl.BlockSpec((MB, K), lambda i, c: (i, 0)),
            pl.BlockSpec((K, 128), lambda i, c: (0, c)),
            pl.BlockSpec((MB, 128), lambda i, c: (i, 0)),
        ],
        out_specs=pl.BlockSpec((MB, 128), lambda i, c: (c * NBN + i, 0)),
        out_shape=jax.ShapeDtypeStruct((C * N, 128), _f32),
    )(xin, W, dinv)


def _epi_call(agg, dinv, b, C, relu):
    def body(a_ref, d_ref, b_ref, o_ref):
        r = a_ref[...] * d_ref[...] + b_ref[0]
        o_ref[...] = jnp.maximum(r, 0.0) if relu else r

    return pl.pallas_call(
        body,
        grid=(NBN, C),
        in_specs=[
            pl.BlockSpec((MB, 128), lambda i, c: (c * NBN + i, 0)),
            pl.BlockSpec((MB, 128), lambda i, c: (i, 0)),
            pl.BlockSpec((1, 1, 128), lambda i, c: (c, 0, 0)),
        ],
        out_specs=pl.BlockSpec((MB, 128), lambda i, c: (i, c)),
        out_shape=jax.ShapeDtypeStruct((N, C * 128), _f32),
    )(agg, dinv, b.reshape(C, 1, 128))


def kernel(x, edge_index, W1, b1, W2, b2):
    ei = edge_index.astype(jnp.int32)
    src = ei[0]
    dst = ei[1]

    src3 = src.reshape(NS, NB, B)

    degp = _deg_kernel(dst, jnp.zeros((N, 16), _f32))
    dinv = _dinv_call(degp)

    hs1 = _mm_chunked(x, W1, dinv, 4)
    agg1 = _agg4(hs1, src3, dst)
    out1 = _epi_call(agg1, dinv, b1, 4, relu=True)

    hs2 = _mm_chunked(out1, W2, dinv, 2)
    agg2 = _agg2(hs2, src3, dst)
    out2 = _epi_call(agg2, dinv, b2, 2, relu=False)

    return (out2, x, out1, out2)


# speedup vs baseline: 13.4041x; 1.0091x over previous
import functools

import jax
import jax.numpy as jnp
from jax import lax
from jax.experimental import pallas as pl
from jax.experimental.pallas import tpu as pltpu
from jax.experimental.pallas import tpu_sc as plsc

N = 10000
E = 160000
NC = 2
NS = 16
L = 16
RS = 640
B = 80
EPS = E // NS
NB = EPS // B
BD = 40
BD2 = 80
NBD2 = 62
EPW = E // (NC * NS)
MB = 1000
NBN = N // MB

_f32 = jnp.float32
_mesh = plsc.VectorSubcoreMesh(core_axis_name="c", subcore_axis_name="s")


@functools.partial(
    pl.kernel,
    out_type=jax.ShapeDtypeStruct((NC * N, 16), _f32),
    mesh=_mesh,
    scratch_types=[
        pltpu.VMEM_SHARED((N, 16), _f32),
        pltpu.VMEM((BD2,), jnp.int32),
        pltpu.VMEM((BD2,), jnp.int32),
        pltpu.VMEM((BD,), jnp.int32),
        pltpu.VMEM((BD2, 16), _f32),
        pltpu.SemaphoreType.DMA,
        pltpu.SemaphoreType.DMA,
    ],
)
def _deg_kernel(dst_hbm, zeros_hbm, out_hbm, acc, dstb0, dstb1, dstbt,
                ones_v, sem0, sem1):
    c = lax.axis_index("c")
    s = lax.axis_index("s")

    @pl.loop(0, BD2)
    def _(i):
        ones_v[i, :] = jnp.ones((L,), _f32)

    @pl.when(s < NS - 1)
    def _():
        pltpu.sync_copy(zeros_hbm.at[pl.ds(s * RS, RS)],
                        acc.at[pl.ds(s * RS, RS)])

    @pl.when(s == NS - 1)
    def _():
        tail = N - (NS - 1) * RS
        pltpu.sync_copy(zeros_hbm.at[pl.ds((NS - 1) * RS, tail)],
                        acc.at[pl.ds((NS - 1) * RS, tail)])

    plsc.subcore_barrier()

    base = (s * NC + c) * EPW

    def _load(bk, dbuf, sem):
        pltpu.async_copy(dst_hbm.at[pl.ds(base + bk * BD2, BD2)], dbuf, sem)

    def _drain(bk, dbuf, sem):
        pltpu.make_async_copy(dst_hbm.at[pl.ds(base + bk * BD2, BD2)],
                              dbuf, sem).wait()
        pltpu.sync_copy(ones_v, acc.at[dbuf], add=True)

    @pl.loop(0, NBD2)
    def _(bk):
        _load(bk, dstb0, sem0)
        _drain(bk, dstb0, sem0)

    pltpu.sync_copy(dst_hbm.at[pl.ds(base + NBD2 * BD2, BD)], dstbt)
    pltpu.sync_copy(ones_v.at[pl.ds(0, BD)], acc.at[dstbt], add=True)

    plsc.subcore_barrier()

    @pl.when(s < NS - 1)
    def _():
        pltpu.sync_copy(acc.at[pl.ds(s * RS, RS)],
                        out_hbm.at[pl.ds(c * N + s * RS, RS)])

    @pl.when(s == NS - 1)
    def _():
        pltpu.sync_copy(acc.at[pl.ds((NS - 1) * RS, N - (NS - 1) * RS)],
                        out_hbm.at[pl.ds(c * N + (NS - 1) * RS,
                                         N - (NS - 1) * RS)])


def _make_agg_kernel(C):
    @functools.partial(
        pl.kernel,
        out_type=jax.ShapeDtypeStruct((C * N, 128), _f32),
        mesh=_mesh,
        scratch_types=[
            pltpu.VMEM_SHARED((N, 128), _f32),
            pltpu.VMEM((NB, B), jnp.int32),
            pltpu.VMEM((B,), jnp.int32),
            pltpu.VMEM((B,), jnp.int32),
            pltpu.VMEM((B, 128), _f32),
            pltpu.VMEM((B, 128), _f32),
            pltpu.SemaphoreType.DMA,
            pltpu.SemaphoreType.DMA,
        ],
    )
    def _agg(hs_hbm, src_hbm, dst_hbm, out_hbm, acc, srcv, dstb0, dstb1,
             rows0, rows1, sem0, sem1):
        c = lax.axis_index("c")
        s = lax.axis_index("s")

        pltpu.sync_copy(src_hbm.at[s], srcv)

        def _shift(delta):
            @pl.loop(0, NB)
            def _(i):
                for j in range(B // L):
                    sl = pl.ds(j * L, L)
                    srcv[i, sl] = srcv[i, sl] + delta

        _shift(c * N)

        def _gather(b, rbuf, dbuf, sem):
            pltpu.async_copy(hs_hbm.at[srcv.at[b]], rbuf, sem)
            pltpu.async_copy(dst_hbm.at[pl.ds(s * EPS + b * B, B)], dbuf, sem)

        def _drain(b, rbuf, dbuf, sem):
            pltpu.make_async_copy(hs_hbm.at[srcv.at[b]], rbuf, sem).wait()
            pltpu.make_async_copy(dst_hbm.at[pl.ds(s * EPS + b * B, B)],
                                  dbuf, sem).wait()
            pltpu.sync_copy(rbuf, acc.at[dbuf], add=True)

        for p in range(C // NC):
            chunk = c + NC * p
            row0 = chunk * N
            if p > 0:
                _shift(NC * N)
            @pl.when(s < NS - 1)
            def _():
                pltpu.sync_copy(hs_hbm.at[pl.ds(row0 + s * RS, RS)],
                                acc.at[pl.ds(s * RS, RS)])

            @pl.when(s == NS - 1)
            def _():
                tail = N - (NS - 1) * RS
                pltpu.sync_copy(hs_hbm.at[pl.ds(row0 + (NS - 1) * RS, tail)],
                                acc.at[pl.ds((NS - 1) * RS, tail)])

            plsc.subcore_barrier()

            _gather(0, rows0, dstb0, sem0)

            @pl.loop(0, (NB - 1) // 2)
            def _(k):
                b = 2 * k
                _gather(b + 1, rows1, dstb1, sem1)
                _drain(b, rows0, dstb0, sem0)
                _gather(b + 2, rows0, dstb0, sem0)
                _drain(b + 1, rows1, dstb1, sem1)

            _drain(NB - 1, rows0, dstb0, sem0)

            plsc.subcore_barrier()

            @pl.when(s < NS - 1)
            def _():
                pltpu.sync_copy(acc.at[pl.ds(s * RS, RS)],
                                out_hbm.at[pl.ds(row0 + s * RS, RS)])

            @pl.when(s == NS - 1)
            def _():
                tail = N - (NS - 1) * RS
                pltpu.sync_copy(acc.at[pl.ds((NS - 1) * RS, tail)],
                                out_hbm.at[pl.ds(row0 + (NS - 1) * RS, tail)])

            if p + 1 < C // NC:
                plsc.subcore_barrier()

    return _agg


_agg2 = _make_agg_kernel(2)
_agg4 = _make_agg_kernel(4)


def _dinv_call(degp):
    def body(p_ref, o_ref):
        deg = p_ref[0:N, :] + p_ref[N:2 * N, :] + 1.0
        o_ref[...] = jax.lax.rsqrt(jnp.concatenate([deg] * 8, axis=1))

    return pl.pallas_call(
        body, out_shape=jax.ShapeDtypeStruct((N, 128), _f32))(degp)


def _mm1_call(x, W1, dinv):
    def body(x_ref, w_ref, d_ref, o_ref):
        o_ref[...] = jnp.dot(x_ref[...], w_ref[...],
                             preferred_element_type=_f32) * d_ref[...]

    return pl.pallas_call(
        body,
        grid=(NBN, 4),
        in_specs=[
            pl.BlockSpec((MB, 256), lambda i, c: (i, 0)),
            pl.BlockSpec((256, 128), lambda i, c: (0, c)),
            pl.BlockSpec((MB, 128), lambda i, c: (i, 0)),
        ],
        out_specs=pl.BlockSpec((MB, 128), lambda i, c: (c * NBN + i, 0)),
        out_shape=jax.ShapeDtypeStruct((4 * N, 128), _f32),
    )(x, W1, dinv)


def _mm2_fused(agg1, dinv, b1, W2):
    def body(a_ref, d_ref, b_ref, w_ref, out1_ref, hs2_ref):
        d = d_ref[...]
        cols = [jnp.maximum(a_ref[c] * d + b_ref[c], 0.0) for c in range(4)]
        lhs = jnp.concatenate(cols, axis=1)
        out1_ref[...] = lhs
        hs2_ref[...] = jnp.dot(lhs, w_ref[...],
                               preferred_element_type=_f32) * d

    return pl.pallas_call(
        body,
        grid=(NBN, 2),
        in_specs=[
            pl.BlockSpec((4, MB, 128), lambda i, c: (0, i, 0)),
            pl.BlockSpec((MB, 128), lambda i, c: (i, 0)),
            pl.BlockSpec((4, 1, 128), lambda i, c: (0, 0, 0)),
            pl.BlockSpec((512, 128), lambda i, c: (0, c)),
        ],
        out_specs=[
            pl.BlockSpec((MB, 512), lambda i, c: (i, 0)),
            pl.BlockSpec((MB, 128), lambda i, c: (c * NBN + i, 0)),
        ],
        out_shape=[
            jax.ShapeDtypeStruct((N, 512), _f32),
            jax.ShapeDtypeStruct((2 * N, 128), _f32),
        ],
    )(agg1.reshape(4, N, 128), dinv, b1.reshape(4, 1, 128), W2)


def _epi_call(agg, dinv, b, C, relu):
    def body(a_ref, d_ref, b_ref, o_ref):
        r = a_ref[...] * d_ref[...] + b_ref[0]
        o_ref[...] = jnp.maximum(r, 0.0) if relu else r

    return pl.pallas_call(
        body,
        grid=(NBN, C),
        in_specs=[
            pl.BlockSpec((MB, 128), lambda i, c: (c * NBN + i, 0)),
            pl.BlockSpec((MB, 128), lambda i, c: (i, 0)),
            pl.BlockSpec((1, 1, 128), lambda i, c: (c, 0, 0)),
        ],
        out_specs=pl.BlockSpec((MB, 128), lambda i, c: (i, c)),
        out_shape=jax.ShapeDtypeStruct((N, C * 128), _f32),
    )(agg, dinv, b.reshape(C, 1, 128))


def kernel(x, edge_index, W1, b1, W2, b2):
    ei = edge_index.astype(jnp.int32)
    src = ei[0]
    dst = ei[1]

    src3 = src.reshape(NS, NB, B)

    degp = _deg_kernel(dst, jnp.zeros((N, 16), _f32))
    dinv = _dinv_call(degp)

    hs1 = _mm1_call(x, W1, dinv)
    agg1 = _agg4(hs1, src3, dst)
    out1 = _epi_call(agg1, dinv, b1, 4, relu=True)
    _, hs2 = _mm2_fused(agg1, dinv, b1, W2)
    agg2 = _agg2(hs2, src3, dst)
    out2 = _epi_call(agg2, dinv, b2, 2, relu=False)

    return (out2, x, out1, out2)


# speedup vs baseline: 14.9206x; 1.1131x over previous
import functools

import jax
import jax.numpy as jnp
from jax import lax
from jax.experimental import pallas as pl
from jax.experimental.pallas import tpu as pltpu
from jax.experimental.pallas import tpu_sc as plsc

N = 10000
E = 160000
NC = 2
NS = 16
L = 16
RS = 640
B = 80
EPS = E // NS
NB = EPS // B
BD = 40
BD2 = 80
NBD2 = 62
EPW = E // (NC * NS)
MB = 1000
NBN = N // MB

_f32 = jnp.float32
_mesh = plsc.VectorSubcoreMesh(core_axis_name="c", subcore_axis_name="s")


@functools.partial(
    pl.kernel,
    out_type=jax.ShapeDtypeStruct((NC * N, 16), _f32),
    mesh=_mesh,
    scratch_types=[
        pltpu.VMEM_SHARED((N, 16), _f32),
        pltpu.VMEM((BD2,), jnp.int32),
        pltpu.VMEM((BD2,), jnp.int32),
        pltpu.VMEM((BD,), jnp.int32),
        pltpu.VMEM((BD2, 16), _f32),
        pltpu.SemaphoreType.DMA,
        pltpu.SemaphoreType.DMA,
    ],
)
def _deg_kernel(dst_hbm, zeros_hbm, out_hbm, acc, dstb0, dstb1, dstbt,
                ones_v, sem0, sem1):
    c = lax.axis_index("c")
    s = lax.axis_index("s")

    @pl.loop(0, BD2)
    def _(i):
        ones_v[i, :] = jnp.ones((L,), _f32)

    @pl.when(s < NS - 1)
    def _():
        pltpu.sync_copy(zeros_hbm.at[pl.ds(s * RS, RS)],
                        acc.at[pl.ds(s * RS, RS)])

    @pl.when(s == NS - 1)
    def _():
        tail = N - (NS - 1) * RS
        pltpu.sync_copy(zeros_hbm.at[pl.ds((NS - 1) * RS, tail)],
                        acc.at[pl.ds((NS - 1) * RS, tail)])

    plsc.subcore_barrier()

    base = (s * NC + c) * EPW

    def _load(bk, dbuf, sem):
        pltpu.async_copy(dst_hbm.at[pl.ds(base + bk * BD2, BD2)], dbuf, sem)

    def _drain(bk, dbuf, sem):
        pltpu.make_async_copy(dst_hbm.at[pl.ds(base + bk * BD2, BD2)],
                              dbuf, sem).wait()
        pltpu.sync_copy(ones_v, acc.at[dbuf], add=True)

    @pl.loop(0, NBD2)
    def _(bk):
        _load(bk, dstb0, sem0)
        _drain(bk, dstb0, sem0)

    pltpu.sync_copy(dst_hbm.at[pl.ds(base + NBD2 * BD2, BD)], dstbt)
    pltpu.sync_copy(ones_v.at[pl.ds(0, BD)], acc.at[dstbt], add=True)

    plsc.subcore_barrier()

    @pl.when(s < NS - 1)
    def _():
        pltpu.sync_copy(acc.at[pl.ds(s * RS, RS)],
                        out_hbm.at[pl.ds(c * N + s * RS, RS)])

    @pl.when(s == NS - 1)
    def _():
        pltpu.sync_copy(acc.at[pl.ds((NS - 1) * RS, N - (NS - 1) * RS)],
                        out_hbm.at[pl.ds(c * N + (NS - 1) * RS,
                                         N - (NS - 1) * RS)])


def _make_agg_kernel(C):
    @functools.partial(
        pl.kernel,
        out_type=jax.ShapeDtypeStruct((C * N, 128), _f32),
        mesh=_mesh,
        scratch_types=[
            pltpu.VMEM_SHARED((N, 128), _f32),
            pltpu.VMEM((NB, B), jnp.int32),
            pltpu.VMEM((B,), jnp.int32),
            pltpu.VMEM((B,), jnp.int32),
            pltpu.VMEM((B,), jnp.int32),
            pltpu.VMEM((B, 128), _f32),
            pltpu.VMEM((B, 128), _f32),
            pltpu.VMEM((B, 128), _f32),
            pltpu.SemaphoreType.DMA,
            pltpu.SemaphoreType.DMA,
            pltpu.SemaphoreType.DMA,
            pltpu.SemaphoreType.DMA,
            pltpu.SemaphoreType.DMA,
            pltpu.SemaphoreType.DMA,
        ],
    )
    def _agg(hs_hbm, src_hbm, dst_hbm, out_hbm, acc, srcv, dstb0, dstb1,
             dstb2, rows0, rows1, rows2, gsem0, gsem1, gsem2,
             ssem0, ssem1, ssem2):
        c = lax.axis_index("c")
        s = lax.axis_index("s")

        pltpu.sync_copy(src_hbm.at[s], srcv)

        def _shift(delta):
            @pl.loop(0, NB)
            def _(i):
                for j in range(B // L):
                    sl = pl.ds(j * L, L)
                    srcv[i, sl] = srcv[i, sl] + delta

        _shift(c * N)

        bufs = ((rows0, dstb0, gsem0, ssem0),
                (rows1, dstb1, gsem1, ssem1),
                (rows2, dstb2, gsem2, ssem2))

        def _gather(b, i):
            rbuf, dbuf, gsem, _ = bufs[i]
            pltpu.async_copy(hs_hbm.at[srcv.at[b]], rbuf, gsem)
            pltpu.async_copy(dst_hbm.at[pl.ds(s * EPS + b * B, B)], dbuf, gsem)

        def _wait_gather(b, i):
            rbuf, dbuf, gsem, _ = bufs[i]
            pltpu.make_async_copy(hs_hbm.at[srcv.at[b]], rbuf, gsem).wait()
            pltpu.make_async_copy(dst_hbm.at[pl.ds(s * EPS + b * B, B)],
                                  dbuf, gsem).wait()

        def _scatter(i):
            rbuf, dbuf, _, ssem = bufs[i]
            pltpu.async_copy(rbuf, acc.at[dbuf], ssem, add=True)

        def _wait_scatter(i):
            rbuf, dbuf, _, ssem = bufs[i]
            pltpu.make_async_copy(rbuf, acc.at[dbuf], ssem).wait()

        for p in range(C // NC):
            chunk = c + NC * p
            row0 = chunk * N
            if p > 0:
                _shift(NC * N)
            @pl.when(s < NS - 1)
            def _():
                pltpu.sync_copy(hs_hbm.at[pl.ds(row0 + s * RS, RS)],
                                acc.at[pl.ds(s * RS, RS)])

            @pl.when(s == NS - 1)
            def _():
                tail = N - (NS - 1) * RS
                pltpu.sync_copy(hs_hbm.at[pl.ds(row0 + (NS - 1) * RS, tail)],
                                acc.at[pl.ds((NS - 1) * RS, tail)])

            plsc.subcore_barrier()

            _gather(0, 0)
            _gather(1, 1)
            _wait_gather(0, 0); _scatter(0); _gather(2, 2)
            _wait_gather(1, 1); _scatter(1); _wait_scatter(0); _gather(3, 0)
            _wait_gather(2, 2); _scatter(2); _wait_scatter(1); _gather(4, 1)

            @pl.loop(1, (NB - 2) // 3)
            def _(g):
                b = 3 * g
                _wait_gather(b, 0)
                _scatter(0)
                _wait_scatter(2)
                _gather(b + 2, 2)
                _wait_gather(b + 1, 1)
                _scatter(1)
                _wait_scatter(0)
                _gather(b + 3, 0)
                _wait_gather(b + 2, 2)
                _scatter(2)
                _wait_scatter(1)
                _gather(b + 4, 1)

            _wait_gather(NB - 2, 0); _scatter(0); _wait_scatter(2)
            _wait_gather(NB - 1, 1); _scatter(1); _wait_scatter(0)
            _wait_scatter(1)

            plsc.subcore_barrier()

            @pl.when(s < NS - 1)
            def _():
                pltpu.sync_copy(acc.at[pl.ds(s * RS, RS)],
                                out_hbm.at[pl.ds(row0 + s * RS, RS)])

            @pl.when(s == NS - 1)
            def _():
                tail = N - (NS - 1) * RS
                pltpu.sync_copy(acc.at[pl.ds((NS - 1) * RS, tail)],
                                out_hbm.at[pl.ds(row0 + (NS - 1) * RS, tail)])

            if p + 1 < C // NC:
                plsc.subcore_barrier()

    return _agg


_agg2 = _make_agg_kernel(2)
_agg4 = _make_agg_kernel(4)


def _dinv_call(degp):
    def body(p_ref, o_ref):
        deg = p_ref[0:N, :] + p_ref[N:2 * N, :] + 1.0
        o_ref[...] = jax.lax.rsqrt(jnp.concatenate([deg] * 8, axis=1))

    return pl.pallas_call(
        body, out_shape=jax.ShapeDtypeStruct((N, 128), _f32))(degp)


def _mm1_call(x, W1, dinv):
    def body(x_ref, w_ref, d_ref, o_ref):
        o_ref[...] = jnp.dot(x_ref[...], w_ref[...],
                             preferred_element_type=_f32) * d_ref[...]

    return pl.pallas_call(
        body,
        grid=(NBN, 4),
        in_specs=[
            pl.BlockSpec((MB, 256), lambda i, c: (i, 0)),
            pl.BlockSpec((256, 128), lambda i, c: (0, c)),
            pl.BlockSpec((MB, 128), lambda i, c: (i, 0)),
        ],
        out_specs=pl.BlockSpec((MB, 128), lambda i, c: (c * NBN + i, 0)),
        out_shape=jax.ShapeDtypeStruct((4 * N, 128), _f32),
    )(x, W1, dinv)


def _mm2_fused(agg1, dinv, b1, W2):
    def body(a_ref, d_ref, b_ref, w_ref, out1_ref, hs2_ref):
        d = d_ref[...]
        cols = [jnp.maximum(a_ref[c] * d + b_ref[c], 0.0) for c in range(4)]
        lhs = jnp.concatenate(cols, axis=1)
        out1_ref[...] = lhs
        hs2_ref[...] = jnp.dot(lhs, w_ref[...],
                               preferred_element_type=_f32) * d

    return pl.pallas_call(
        body,
        grid=(NBN, 2),
        in_specs=[
            pl.BlockSpec((4, MB, 128), lambda i, c: (0, i, 0)),
            pl.BlockSpec((MB, 128), lambda i, c: (i, 0)),
            pl.BlockSpec((4, 1, 128), lambda i, c: (0, 0, 0)),
            pl.BlockSpec((512, 128), lambda i, c: (0, c)),
        ],
        out_specs=[
            pl.BlockSpec((MB, 512), lambda i, c: (i, 0)),
            pl.BlockSpec((MB, 128), lambda i, c: (c * NBN + i, 0)),
        ],
        out_shape=[
            jax.ShapeDtypeStruct((N, 512), _f32),
            jax.ShapeDtypeStruct((2 * N, 128), _f32),
        ],
    )(agg1.reshape(4, N, 128), dinv, b1.reshape(4, 1, 128), W2)


def _epi_call(agg, dinv, b, C, relu):
    def body(a_ref, d_ref, b_ref, o_ref):
        r = a_ref[...] * d_ref[...] + b_ref[0]
        o_ref[...] = jnp.maximum(r, 0.0) if relu else r

    return pl.pallas_call(
        body,
        grid=(NBN, C),
        in_specs=[
            pl.BlockSpec((MB, 128), lambda i, c: (c * NBN + i, 0)),
            pl.BlockSpec((MB, 128), lambda i, c: (i, 0)),
            pl.BlockSpec((1, 1, 128), lambda i, c: (c, 0, 0)),
        ],
        out_specs=pl.BlockSpec((MB, 128), lambda i, c: (i, c)),
        out_shape=jax.ShapeDtypeStruct((N, C * 128), _f32),
    )(agg, dinv, b.reshape(C, 1, 128))


def kernel(x, edge_index, W1, b1, W2, b2):
    ei = edge_index.astype(jnp.int32)
    src = ei[0]
    dst = ei[1]

    src3 = src.reshape(NS, NB, B)

    degp = _deg_kernel(dst, jnp.zeros((N, 16), _f32))
    dinv = _dinv_call(degp)

    hs1 = _mm1_call(x, W1, dinv)
    agg1 = _agg4(hs1, src3, dst)
    out1 = _epi_call(agg1, dinv, b1, 4, relu=True)
    _, hs2 = _mm2_fused(agg1, dinv, b1, W2)
    agg2 = _agg2(hs2, src3, dst)
    out2 = _epi_call(agg2, dinv, b2, 2, relu=False)

    return (out2, x, out1, out2)
